# Initial kernel scaffold; baseline (speedup 1.0000x reference)
import jax
import jax.numpy as jnp
from jax.experimental import pallas as pl


def kernel(x, slopes, intercepts, bias):
    raise NotImplementedError("write your pallas kernel here")


# speedup vs baseline: 761.1130x; 761.1130x over previous
import jax
import jax.numpy as jnp
from jax.experimental import pallas as pl

INPUT_MIN, INPUT_MAX = 0.0, 1.0


def _spline_body(x_ref, w_ref, bias_ref, out_ref):
    k = w_ref.shape[0]
    xv = x_ref[:]
    x_norm = (xv - INPUT_MIN) / (INPUT_MAX - INPUT_MIN)
    idx = jnp.clip(jnp.floor(x_norm * k).astype(jnp.int32), 0, k - 1)
    acc = jnp.zeros((xv.shape[0], w_ref.shape[2]), jnp.float32)
    for kk in range(k):
        mask = (idx == kk).astype(jnp.float32)
        act = jnp.concatenate([xv * mask, mask], axis=1)
        acc = acc + jnp.dot(act, w_ref[kk],
                            preferred_element_type=jnp.float32)
    out_ref[:] = acc + bias_ref[:]


def kernel(x, slopes, intercepts, bias):
    b, in_dim = x.shape
    out_dim, _, k = slopes.shape
    s_t = jnp.transpose(slopes, (2, 1, 0))
    t_t = jnp.transpose(intercepts, (2, 1, 0))
    w = jnp.concatenate([s_t, t_t], axis=1)
    bias2d = bias.reshape(1, out_dim)

    return pl.pallas_call(
        _spline_body,
        out_shape=jax.ShapeDtypeStruct((b, out_dim), jnp.float32),
    )(x, w, bias2d)
